# baseline (speedup 1.0000x reference)
import jax
import jax.numpy as jnp
from jax.experimental import pallas as pl
from jax.experimental.pallas import tpu as pltpu

_L = 16
_ROWS = 512
_COLS = 128


def _gate_body(idx_smem, idx_vmem, vals_ref, out_ref, rows_v):
    for i in range(_L):
        rows_v[pl.ds(i, 1), :] = vals_ref[pl.ds(idx_smem[i] // _COLS, 1), :]
    lane = jax.lax.broadcasted_iota(jnp.int32, (_L, _COLS), 1)
    col = idx_vmem[...] % _COLS
    picked = jnp.where(lane == col, rows_v[...], 1.0)
    acc = picked[:8, :] * picked[8:, :]
    for sh in (4, 2, 1):
        acc = acc * pltpu.roll(acc, sh, 0)
    acc = acc[0:1, :]
    for sh in (64, 32, 16, 8, 4, 2, 1):
        acc = acc * pltpu.roll(acc, sh, 1)
    out_ref[0] = acc[0, 0]


@jax.jit
def _gate(vals, idx):
    return pl.pallas_call(
        _gate_body,
        in_specs=[
            pl.BlockSpec(memory_space=pltpu.SMEM),
            pl.BlockSpec(memory_space=pltpu.VMEM),
            pl.BlockSpec((_ROWS, _COLS), lambda: (0, 0)),
        ],
        out_specs=pl.BlockSpec(memory_space=pltpu.SMEM),
        out_shape=jax.ShapeDtypeStruct((1,), jnp.float32),
        scratch_shapes=[
            pltpu.VMEM((_L, _COLS), jnp.float32),
        ],
    )(idx, idx.reshape(_L, 1), vals.reshape(_ROWS, _COLS))


def kernel(input_values, input_idxs):
    out = _gate(input_values, input_idxs.astype(jnp.int32))
    return out.reshape(())


# speedup vs baseline: 1.6889x; 1.6889x over previous
import functools

import jax
import jax.numpy as jnp
from jax.experimental import pallas as pl
from jax.experimental.pallas import tpu as pltpu

_L = 16
_ROWS = 512
_COLS = 128


def _gate_body(idx_smem, vals_ref, out_ref):
    lane = jax.lax.broadcasted_iota(jnp.int32, (1, _COLS), 1)
    parts = []
    for i in range(_L):
        idx = idx_smem[i]
        vrow = vals_ref[pl.ds(idx // _COLS, 1), :]
        rolled = pltpu.roll(vrow, (i - idx) % _COLS, 1)
        parts.append(jnp.where(lane == i, rolled, 1.0))
    while len(parts) > 1:
        parts = [a * b for a, b in zip(parts[::2], parts[1::2])]
    acc = parts[0]
    for sh in (8, 4, 2, 1):
        acc = acc * pltpu.roll(acc, _COLS - sh, 1)
    out_ref[0] = acc[0, 0]


@jax.jit
def _gate(vals, idx):
    return pl.pallas_call(
        _gate_body,
        in_specs=[
            pl.BlockSpec(memory_space=pltpu.SMEM),
            pl.BlockSpec((_ROWS, _COLS), lambda: (0, 0)),
        ],
        out_specs=pl.BlockSpec(memory_space=pltpu.SMEM),
        out_shape=jax.ShapeDtypeStruct((1,), jnp.float32),
    )(idx, vals.reshape(_ROWS, _COLS))


def kernel(input_values, input_idxs):
    out = _gate(input_values, input_idxs.astype(jnp.int32))
    return out.reshape(())
